# baseline (speedup 1.0000x reference)
import functools

import jax
import jax.numpy as jnp
from jax import lax
from jax.experimental import pallas as pl
from jax.experimental.pallas import tpu as pltpu
from jax.experimental.pallas import tpu_sc as plsc

_B = 4096
_F = 26
_V = 100000
_D = 32

_NC = 2
_NS = 16
_NW = _NC * _NS
_ROWS = _B * _F
_RPW = _ROWS // _NW
_CHUNK = 128
_NCHUNK = _RPW // _CHUNK


_KB = _F * _D // 8
_NTILE = 4


def _sc_gather_body(tbl, idx_hbm, off_hbm, out_hbm, idx_v, off_v, rows_v, tile_v,
                    sem, dsem):
    wid = lax.axis_index("s") * _NC + lax.axis_index("c")
    base = wid * _RPW
    pltpu.sync_copy(idx_hbm.at[pl.ds(base, _RPW)], idx_v)
    pltpu.sync_copy(off_hbm, off_v)

    def _add(i, carry):
        for u in range(4):
            s = pl.ds((i * 4 + u) * 16, 16)
            idx_v[s] = idx_v[s] + off_v[s]
        return carry

    lax.fori_loop(0, _RPW // 64, _add, 0)

    def _fire(c, carry):
        pltpu.make_async_copy(
            tbl.at[idx_v.at[pl.ds(c * _CHUNK, _CHUNK)]],
            rows_v.at[pl.ds(c * _CHUNK, _CHUNK)],
            sem,
        ).start()
        return carry

    lax.fori_loop(0, _NCHUNK, _fire, 0)

    def _drain(c, carry):
        pltpu.make_async_copy(
            tbl.at[idx_v.at[pl.ds(c * _CHUNK, _CHUNK)]],
            rows_v.at[pl.ds(c * _CHUNK, _CHUNK)],
            sem,
        ).wait()
        return carry

    lax.fori_loop(0, _NCHUNK, _drain, 0)

    i26 = lax.iota(jnp.int32, 16) * _F

    def _repack(kb, carry):
        slot = lax.rem(kb, _NTILE)

        @pl.when(kb >= _NTILE)
        def _():
            pltpu.make_async_copy(tile_v.at[slot], out_hbm.at[kb - _NTILE, wid],
                                  dsem).wait()

        for kr in range(8):
            k = kb * 8 + kr
            kfd = lax.div(k, _D)
            kc = lax.rem(k, _D)
            for c in range(8):
                row_idx = i26 + (kfd + c * 16 * _F)
                col_idx = jnp.broadcast_to(kc, (16,))
                v = plsc.load_gather(rows_v, [row_idx, col_idx])
                tile_v[slot, kr, pl.ds(c * 16, 16)] = v
        pltpu.make_async_copy(tile_v.at[slot], out_hbm.at[kb, wid], dsem).start()
        return carry

    lax.fori_loop(0, _KB, _repack, 0)

    def _dr(i, carry):
        kb = _KB - _NTILE + i
        slot = lax.rem(kb, _NTILE)
        pltpu.make_async_copy(tile_v.at[slot], out_hbm.at[kb, wid], dsem).wait()
        return carry

    lax.fori_loop(0, _NTILE, _dr, 0)


@functools.cache
def _sc_gather():
    mesh = plsc.VectorSubcoreMesh(core_axis_name="c", subcore_axis_name="s",
                                  num_cores=_NC, num_subcores=_NS)
    return pl.kernel(
        _sc_gather_body,
        out_type=jax.ShapeDtypeStruct((_KB, _NW, 8, 128), jnp.float32),
        mesh=mesh,
        scratch_types=[
            pltpu.VMEM((_RPW,), jnp.int32),
            pltpu.VMEM((_RPW,), jnp.int32),
            pltpu.VMEM((_RPW, _D), jnp.float32),
            pltpu.VMEM((_NTILE, 8, 128), jnp.float32),
            pltpu.SemaphoreType.DMA,
            pltpu.SemaphoreType.DMA,
        ],
        compiler_params=pltpu.CompilerParams(use_tc_tiling_on_sc=False, needs_layout_passes=False),
    )


_BB = 512
_NBLK = _B // _BB
_NFEAT = _F + 1


def _tc_body(xT, emb4, w1t, b1, w2t, b2, w3t, b3,
             ow1t, ob1, ow2t, ob2, ow3t, ob3, out_ref):
    f32 = jnp.float32
    d = jnp.maximum(jnp.dot(w1t[...], xT[...], preferred_element_type=f32) + b1[...], 0.0)
    d = jnp.maximum(jnp.dot(w2t[...], d, preferred_element_type=f32) + b2[...], 0.0)
    d = jnp.maximum(jnp.dot(w3t[...], d, preferred_element_type=f32) + b3[...], 0.0)

    x4 = emb4[...]
    embt = jnp.concatenate(
        [x4[:, i].reshape(_F * _D, 128) for i in range(_BB // 128)], axis=1)
    ct = jnp.concatenate([d, embt], axis=0)

    pieces = [d]
    for f in range(_NFEAT - 1):
        g = _NFEAT - 1 - f
        e = ct[32 * f:32 * (f + 1), :]
        rest = ct[32 * (f + 1):, :].reshape(g, 32, _BB)
        pieces.append(jnp.sum(rest * e[None], axis=1))
    x = jnp.concatenate(pieces, axis=0)

    o = jnp.maximum(jnp.dot(ow1t[...], x, preferred_element_type=f32) + ob1[...], 0.0)
    o = jnp.maximum(jnp.dot(ow2t[...], o, preferred_element_type=f32) + ob2[...], 0.0)
    out_ref[...] = jnp.dot(ow3t[...], o, preferred_element_type=f32) + ob3[...]


def _full(shape):
    return pl.BlockSpec(shape, lambda j: (0, 0))


_tc_forward = pl.pallas_call(
    _tc_body,
    grid=(_NBLK,),
    in_specs=[
        pl.BlockSpec((13, _BB), lambda j: (0, j)),
        pl.BlockSpec((_KB, _BB // 128, 8, 128), lambda j: (0, j, 0, 0)),
        _full((512, 13)), _full((512, 1)),
        _full((256, 512)), _full((256, 1)),
        _full((32, 256)), _full((32, 1)),
        _full((512, 383)), _full((512, 1)),
        _full((256, 512)), _full((256, 1)),
        _full((1, 256)), _full((1, 1)),
    ],
    out_specs=pl.BlockSpec((1, _BB), lambda j: (0, j)),
    out_shape=jax.ShapeDtypeStruct((1, _B), jnp.float32),
)


def kernel(dense_features, sparse_indices, tables, dW1, db1, dW2, db2, dW3, db3,
           oW1, ob1, oW2, ob2, oW3, ob3):
    tbl_flat = tables.reshape(_F * _V, _D)
    idx_flat = sparse_indices.reshape(-1)
    offs = jnp.tile(jnp.arange(_F, dtype=jnp.int32) * _V, _RPW // _F)

    emb4 = _sc_gather()(tbl_flat, idx_flat, offs)

    out_t = _tc_forward(
        dense_features.T, emb4,
        dW1.T, db1.reshape(-1, 1), dW2.T, db2.reshape(-1, 1),
        dW3.T, db3.reshape(-1, 1),
        oW1.T, ob1.reshape(-1, 1), oW2.T, ob2.reshape(-1, 1),
        oW3.T, ob3.reshape(-1, 1),
    )
    return out_t.reshape(_B, 1)


# speedup vs baseline: 1.5502x; 1.5502x over previous
import functools

import jax
import jax.numpy as jnp
from jax import lax
from jax.experimental import pallas as pl
from jax.experimental.pallas import tpu as pltpu
from jax.experimental.pallas import tpu_sc as plsc

_B = 4096
_F = 26
_V = 100000
_D = 32

_NC = 2
_NS = 16
_NW = _NC * _NS
_ROWS = _B * _F
_RPW = _ROWS // _NW
_BPW = _B // _NW
_CHUNK = 128
_HALF = _RPW // 2
_NCH = _HALF // _CHUNK
_KB = _F * _D // 8

_W = 25600
_JC = 3200
_NJC = _W // _JC
_TAIL = _V - 3 * _W - (_NJC - 1) * _JC


def _relayout_body(t_hbm, out_ref, bufs, tmp, sem):
    f = pl.program_id(0)
    jc = pl.program_id(1)

    def _fire(fi):
        pltpu.make_async_copy(t_hbm.at[fi], bufs.at[lax.rem(fi, 2)],
                              sem.at[lax.rem(fi, 2)]).start()

    def _wait(fi):
        pltpu.make_async_copy(t_hbm.at[fi], bufs.at[lax.rem(fi, 2)],
                              sem.at[lax.rem(fi, 2)]).wait()

    @pl.when(jnp.logical_and(f == 0, jc == 0))
    def _():
        _fire(0)

    @pl.when(jc == 0)
    def _():
        _wait(f)

        @pl.when(f + 1 < _F)
        def _():
            _fire(f + 1)

    par = lax.rem(f, 2)

    @pl.when(jc != _NJC - 1)
    def _():
        tmp[...] = bufs[par, :, pl.ds(3 * _W + jc * _JC, _JC)]

    @pl.when(jc == _NJC - 1)
    def _():
        tmp[:, pl.ds(0, _TAIL)] = bufs[par, :, pl.ds(3 * _W + (_NJC - 1) * _JC, _TAIL)]

    pieces = [jnp.transpose(bufs[par, :, pl.ds(q * _W + jc * _JC, _JC)])
              for q in range(3)]
    pieces.append(jnp.transpose(tmp[...]))
    out_ref[...] = jnp.concatenate(pieces, axis=1)


_tbl_relayout = pl.pallas_call(
    _relayout_body,
    grid=(_F, _NJC),
    in_specs=[pl.BlockSpec(memory_space=pl.ANY)],
    out_specs=pl.BlockSpec((_JC, 128), lambda f, jc: (f * _NJC + jc, 0)),
    out_shape=jax.ShapeDtypeStruct((_F * _W, 128), jnp.float32),
    scratch_shapes=[
        pltpu.VMEM((2, _D, _V), jnp.float32),
        pltpu.VMEM((_D, _JC), jnp.float32),
        pltpu.SemaphoreType.DMA((2,)),
    ],
)

_TROWS = _F * _W


def _sc_gather_body(tbl, idx_hbm, out_hbm, idx_v, colb_v, chunk0_v, chunk1_v,
                    ct_v, sem0, sem1):
    wid = lax.axis_index("s") * _NC + lax.axis_index("c")
    base = wid * _RPW
    pltpu.sync_copy(idx_hbm.at[pl.ds(base, _RPW)], idx_v)

    i16 = lax.iota(jnp.int32, 16)

    def _prep(i, carry):
        for u in range(4):
            o = (i * 4 + u) * 16
            s = pl.ds(o, 16)
            f = lax.rem(i16 + o, _F)
            raw = idx_v[s]
            idx_v[s] = f * _W + lax.rem(raw, _W)
            colb_v[s] = lax.shift_left(lax.div(raw, _W), 5)
        return carry

    lax.fori_loop(0, _RPW // 64, _prep, 0)

    def _fire(h, c, buf, sem):
        pltpu.make_async_copy(
            tbl.at[idx_v.at[pl.ds(h * _HALF + c * _CHUNK, _CHUNK)]],
            buf, sem,
        ).start()

    def _wait(h, c, buf, sem):
        pltpu.make_async_copy(
            tbl.at[idx_v.at[pl.ds(h * _HALF + c * _CHUNK, _CHUNK)]],
            buf, sem,
        ).wait()

    def _extract(h, c, buf):
        def _g(g, carry):
            r16 = i16 + (h * _HALF + c * _CHUNK + g * 16)
            rloc = i16 + g * 16
            colb = plsc.load_gather(colb_v, [r16])
            f4 = lax.rem(r16, _F) * 4
            bl = lax.div(r16, _F) - (64 * h)

            def _j(j4, carry2):
                for u in range(4):
                    j = j4 * 4 + u
                    v = plsc.load_gather(buf, [rloc, colb + j])
                    plsc.store_scatter(
                        ct_v,
                        [f4 + lax.shift_right_logical(j, 3),
                         jnp.broadcast_to(lax.bitwise_and(j, 7), (16,)),
                         bl],
                        v)
                return carry2

            lax.fori_loop(0, _D // 4, _j, 0)
            return carry

        lax.fori_loop(0, _CHUNK // 16, _g, 0)

    for h in range(2):
        bufs = (chunk0_v, chunk1_v)
        sems = (sem0, sem1)
        _fire(h, 0, bufs[0], sems[0])
        for c in range(_NCH):
            if c + 1 < _NCH:
                _fire(h, c + 1, bufs[(c + 1) % 2], sems[(c + 1) % 2])
            _wait(h, c, bufs[c % 2], sems[c % 2])
            _extract(h, c, bufs[c % 2])
        pltpu.sync_copy(ct_v, out_hbm.at[:, wid, :, pl.ds(64 * h, 64)])


@functools.cache
def _sc_gather():
    mesh = plsc.VectorSubcoreMesh(core_axis_name="c", subcore_axis_name="s",
                                  num_cores=_NC, num_subcores=_NS)
    return pl.kernel(
        _sc_gather_body,
        out_type=jax.ShapeDtypeStruct((_KB, _NW, 8, 128), jnp.float32),
        mesh=mesh,
        scratch_types=[
            pltpu.VMEM((_RPW,), jnp.int32),
            pltpu.VMEM((_RPW,), jnp.int32),
            pltpu.VMEM((_CHUNK, 128), jnp.float32),
            pltpu.VMEM((_CHUNK, 128), jnp.float32),
            pltpu.VMEM((_KB, 8, 64), jnp.float32),
            pltpu.SemaphoreType.DMA,
            pltpu.SemaphoreType.DMA,
        ],
        compiler_params=pltpu.CompilerParams(use_tc_tiling_on_sc=False,
                                             needs_layout_passes=False),
    )


_BB = 512
_NBLK = _B // _BB
_NFEAT = _F + 1


def _tc_body(xT, emb4, w1t, b1, w2t, b2, w3t, b3,
             ow1t, ob1, ow2t, ob2, ow3t, ob3, out_ref):
    f32 = jnp.float32
    d = jnp.maximum(jnp.dot(w1t[...], xT[...], preferred_element_type=f32) + b1[...], 0.0)
    d = jnp.maximum(jnp.dot(w2t[...], d, preferred_element_type=f32) + b2[...], 0.0)
    d = jnp.maximum(jnp.dot(w3t[...], d, preferred_element_type=f32) + b3[...], 0.0)

    x4 = emb4[...]
    embt = jnp.concatenate(
        [x4[:, i].reshape(_F * _D, 128) for i in range(_BB // 128)], axis=1)
    ct = jnp.concatenate([d, embt], axis=0)

    pieces = [d]
    for f in range(_NFEAT - 1):
        g = _NFEAT - 1 - f
        e = ct[32 * f:32 * (f + 1), :]
        rest = ct[32 * (f + 1):, :].reshape(g, 32, _BB)
        pieces.append(jnp.sum(rest * e[None], axis=1))
    x = jnp.concatenate(pieces, axis=0)

    o = jnp.maximum(jnp.dot(ow1t[...], x, preferred_element_type=f32) + ob1[...], 0.0)
    o = jnp.maximum(jnp.dot(ow2t[...], o, preferred_element_type=f32) + ob2[...], 0.0)
    out_ref[...] = jnp.dot(ow3t[...], o, preferred_element_type=f32) + ob3[...]


def _full(shape):
    return pl.BlockSpec(shape, lambda j: (0, 0))


_tc_forward = pl.pallas_call(
    _tc_body,
    grid=(_NBLK,),
    in_specs=[
        pl.BlockSpec((13, _BB), lambda j: (0, j)),
        pl.BlockSpec((_KB, _BB // 128, 8, 128), lambda j: (0, j, 0, 0)),
        _full((512, 13)), _full((512, 1)),
        _full((256, 512)), _full((256, 1)),
        _full((32, 256)), _full((32, 1)),
        _full((512, 383)), _full((512, 1)),
        _full((256, 512)), _full((256, 1)),
        _full((1, 256)), _full((1, 1)),
    ],
    out_specs=pl.BlockSpec((1, _BB), lambda j: (0, j)),
    out_shape=jax.ShapeDtypeStruct((1, _B), jnp.float32),
)


def kernel(dense_features, sparse_indices, tables, dW1, db1, dW2, db2, dW3, db3,
           oW1, ob1, oW2, ob2, oW3, ob3):
    tbl4 = _tbl_relayout(tables.transpose(0, 2, 1))
    idx_flat = sparse_indices.reshape(-1)

    emb4 = _sc_gather()(tbl4, idx_flat)

    out_t = _tc_forward(
        dense_features.T, emb4,
        dW1.T, db1.reshape(-1, 1), dW2.T, db2.reshape(-1, 1),
        dW3.T, db3.reshape(-1, 1),
        oW1.T, ob1.reshape(-1, 1), oW2.T, ob2.reshape(-1, 1),
        oW3.T, ob3.reshape(-1, 1),
    )
    return out_t.reshape(_B, 1)
